# Initial kernel scaffold; baseline (speedup 1.0000x reference)
import jax
import jax.numpy as jnp
from jax.experimental import pallas as pl


def kernel(nodes, features, neigh1, neigh2, local_weight):
    raise NotImplementedError("write your pallas kernel here")


# speedup vs baseline: 7.7311x; 7.7311x over previous
import functools

import jax
import jax.numpy as jnp
from jax import lax
from jax.experimental import pallas as pl
from jax.experimental.pallas import tpu as pltpu
from jax.experimental.pallas import tpu_sc as plsc

_NC = 2
_NS = 16
_NW = _NC * _NS
_LANES = 16
_NBUF = 4


def _sc_aggregate(nodes_flat, features, nbr):
    n_seeds = nodes_flat.shape[0]
    num_nodes, d = features.shape
    s1, s2 = 25, 10
    ch = d // _LANES
    per_w = n_seeds // _NW
    assert n_seeds % (_NW * 8) == 0
    r_unroll = 5
    assert s1 % r_unroll == 0

    n_groups = s2 + 1

    def body(nodes_ref, feat_ref, nbr_ref,
             agg1_ref, agg2_ref, self_ref,
             seeds_v, comb_v, nm_a, nm_b, self_v, o1_v, o2_v, fbuf,
             sem_self, sem_comb, sem_a, sem_b, *fsems):
        wid = lax.axis_index("s") * _NC + lax.axis_index("c")
        base = wid * per_w

        pltpu.sync_copy(nodes_ref.at[pl.ds(base, per_w)], seeds_v)
        cp_self = pltpu.make_async_copy(feat_ref.at[seeds_v], self_v, sem_self)
        cp_self.start()
        cp_comb = pltpu.make_async_copy(nbr_ref.at[seeds_v], comb_v, sem_comb)
        cp_comb.start()
        cp_comb.wait()
        cp_self.wait()
        pltpu.sync_copy(self_v, self_ref.at[pl.ds(base, per_w)])

        inv1 = jnp.float32(1.0 / s1)
        inv2 = jnp.float32(1.0 / (s1 * s2))

        def feat_desc(seed, g, nmx):
            idxr = (comb_v.at[seed, pl.ds(0, s1)] if g == 0
                    else nmx.at[g - 1, pl.ds(0, s1)])
            return pltpu.make_async_copy(feat_ref.at[idxr], fbuf.at[g],
                                         fsems[g])

        def nm_desc(seed, nmx, sem):
            return pltpu.make_async_copy(
                nbr_ref.at[comb_v.at[seed, pl.ds(32, s2)]], nmx, sem)

        def accum(slot):
            def rbody(rr, accs):
                r0 = rr * r_unroll
                out = list(accs)
                for dr in range(r_unroll):
                    for c in range(ch):
                        out[c] = out[c] + fbuf[slot, r0 + dr,
                                               pl.ds(c * _LANES, _LANES)]
                return out
            init = [jnp.zeros((_LANES,), jnp.float32)] * ch
            return lax.fori_loop(0, s1 // r_unroll, rbody, init)

        def half(seed, nmx, extras):
            a1 = None
            acc2 = [jnp.zeros((_LANES,), jnp.float32)] * ch
            for g in range(n_groups):
                feat_desc(seed, g, nmx).wait()
                sums = accum(g)
                if g == 0:
                    a1 = sums
                else:
                    acc2 = [x + y for x, y in zip(acc2, sums)]
                if g + 4 < n_groups:
                    feat_desc(seed, g + 4, nmx).start()
                elif g in extras:
                    extras[g]()
            for c in range(ch):
                o1_v[seed, pl.ds(c * _LANES, _LANES)] = a1[c] * inv1
                o2_v[seed, pl.ds(c * _LANES, _LANES)] = acc2[c] * inv2

        def next_fires(nseed, nmx, sem):
            return {
                7: lambda: feat_desc(nseed, 0, nmx).start(),
                8: lambda: (nm_desc(nseed, nmx, sem).wait(),
                            feat_desc(nseed, 1, nmx).start()),
                9: lambda: feat_desc(nseed, 2, nmx).start(),
                10: lambda: feat_desc(nseed, 3, nmx).start(),
            }

        nm_desc(0, nm_a, sem_a).start()
        nm_desc(1, nm_b, sem_b).start()
        nm_desc(0, nm_a, sem_a).wait()
        for g in range(4):
            feat_desc(0, g, nm_a).start()

        def pair_body(k, carry):
            a = 2 * k
            b = a + 1
            a2 = jnp.minimum(a + 2, per_w - 1)
            b2 = jnp.minimum(b + 2, per_w - 1)
            half(a, nm_a, next_fires(b, nm_b, sem_b))
            nm_desc(a2, nm_a, sem_a).start()
            half(b, nm_b, next_fires(a2, nm_a, sem_a))
            nm_desc(b2, nm_b, sem_b).start()
            return carry

        lax.fori_loop(0, per_w // 2, pair_body, 0)
        last = per_w - 1
        for g in range(4):
            feat_desc(last, g, nm_a).wait()
        nm_desc(last, nm_b, sem_b).wait()
        pltpu.sync_copy(o1_v, agg1_ref.at[pl.ds(base, per_w)])
        pltpu.sync_copy(o2_v, agg2_ref.at[pl.ds(base, per_w)])

    mesh = plsc.VectorSubcoreMesh(
        core_axis_name="c", subcore_axis_name="s",
        num_cores=_NC, num_subcores=_NS)
    f32 = jnp.float32
    out_type = [jax.ShapeDtypeStruct((n_seeds, d), f32)] * 3
    scratch = [
        pltpu.VMEM((per_w,), jnp.int32),
        pltpu.VMEM((per_w, 128), jnp.int32),
        pltpu.VMEM((s2, 128), jnp.int32),
        pltpu.VMEM((s2, 128), jnp.int32),
        pltpu.VMEM((per_w, d), f32),
        pltpu.VMEM((per_w, d), f32),
        pltpu.VMEM((per_w, d), f32),
        pltpu.VMEM((n_groups, s1, d), f32),
        pltpu.SemaphoreType.DMA,
        pltpu.SemaphoreType.DMA,
        pltpu.SemaphoreType.DMA,
        pltpu.SemaphoreType.DMA,
    ] + [pltpu.SemaphoreType.DMA] * n_groups
    fn = pl.kernel(body, out_type=out_type, mesh=mesh, scratch_types=scratch)
    return fn(nodes_flat, features, nbr)


def _project(agg1, agg2, self_feats, local_weight):
    n, d = agg1.shape
    c_clips, k_dim, _ = local_weight.shape

    def tc_body(x1_ref, x2_ref, x3_ref, w_ref, o_ref):
        for s_idx, xr in enumerate((x1_ref, x2_ref, x3_ref)):
            x = xr[...]
            for c in range(c_clips):
                y = lax.dot_general(
                    x, w_ref[c], (((1,), (1,)), ((), ())),
                    preferred_element_type=jnp.float32)
                o_ref[:, c, pl.ds(s_idx * k_dim, k_dim)] = jnp.maximum(y, 0.0)

    return pl.pallas_call(
        tc_body,
        out_shape=jax.ShapeDtypeStruct((n, c_clips, 3 * k_dim), jnp.float32),
    )(agg1, agg2, self_feats, local_weight)


@jax.jit
def kernel(nodes, features, neigh1, neigh2, local_weight):
    b, l = nodes.shape
    nodes_flat = nodes.reshape(-1)
    s1, s2 = neigh1.shape[1], neigh2.shape[1]
    nbr = jnp.concatenate(
        [neigh1, jnp.zeros((neigh1.shape[0], 32 - s1), jnp.int32),
         neigh2, jnp.zeros((neigh1.shape[0], 128 - 32 - s2), jnp.int32)],
        axis=1)
    agg1, agg2, self_feats = _sc_aggregate(nodes_flat, features, nbr)
    out = _project(agg1, agg2, self_feats, local_weight)
    c_clips, k_dim, _ = local_weight.shape
    return out.reshape(b, l, c_clips, 3 * k_dim)
